# baseline (speedup 1.0000x reference)
import functools

import numpy as np
import jax
import jax.numpy as jnp
from jax import lax
from jax.experimental import pallas as pl
from jax.experimental.pallas import tpu as pltpu
from jax.experimental.pallas import tpu_sc as plsc

N, E, K = 10000, 320000, 64
DN, DE, DG = 128, 16, 128
F32 = jnp.float32

NC, NS = 2, 16
NW = NC * NS

EPW = E // NW
G1_CB = 1000
G1_SUB = 40
G1_NSUB = G1_CB // G1_SUB
G1_NCH = EPW // G1_CB

NPAD = 10240
NPW = NPAD // NW
G2_NPC = 8
G2_IDXC = G2_NPC * K
G2_SUB = 128
G2_NSUB = G2_IDXC // G2_SUB

_PREC = lax.Precision.DEFAULT

_I8 = np.eye(8, dtype=np.float32)
BD_MASK = np.kron(_I8, np.ones((16, 16), np.float32))
BD_MEAN = np.kron(_I8, np.full((16, 16), 1.0 / 16, np.float32))
M_MEAN128 = np.full((128, 128), 1.0 / 128, np.float32)
FOLD16 = np.tile(np.eye(16, dtype=np.float32), (8, 1))


def _dot(a, b):
    return jnp.dot(a, b, preferred_element_type=F32, precision=_PREC)


def _prep_body(v_ref, we_ref, g_ref, be_ref, wn_ref, bn_ref, mask_ref,
               ps_ref, pr_ref, ce_ref, cn_ref, bdw_ref):
    v = v_ref[...]
    ps_ref[...] = _dot(v, we_ref[16:144, :])
    pr_ref[...] = _dot(v, we_ref[144:272, :])
    ce_ref[...] = _dot(g_ref[...], we_ref[272:400, :]) + be_ref[...]
    cn_ref[...] = _dot(g_ref[...], wn_ref[144:272, :]) + bn_ref[...]
    wee = we_ref[0:16, :]
    col = jnp.concatenate([wee] * 8, axis=0)
    bdw_ref[...] = jnp.concatenate([col] * 8, axis=1) * mask_ref[...]


def _prep(v, W_edge, g, b_edge, W_node, b_node):
    blk = 2000
    grid = (N // blk,)
    return pl.pallas_call(
        _prep_body,
        grid=grid,
        in_specs=[
            pl.BlockSpec((blk, DN), lambda t: (t, 0)),
            pl.BlockSpec((400, DE), lambda t: (0, 0)),
            pl.BlockSpec((1, DG), lambda t: (0, 0)),
            pl.BlockSpec((1, DE), lambda t: (0, 0)),
            pl.BlockSpec((272, DN), lambda t: (0, 0)),
            pl.BlockSpec((1, DN), lambda t: (0, 0)),
            pl.BlockSpec((DN, DN), lambda t: (0, 0)),
        ],
        out_specs=[
            pl.BlockSpec((blk, DE), lambda t: (t, 0)),
            pl.BlockSpec((blk, DE), lambda t: (t, 0)),
            pl.BlockSpec((1, DE), lambda t: (0, 0)),
            pl.BlockSpec((1, DN), lambda t: (0, 0)),
            pl.BlockSpec((DN, DN), lambda t: (0, 0)),
        ],
        out_shape=[
            jax.ShapeDtypeStruct((N, DE), F32),
            jax.ShapeDtypeStruct((N, DE), F32),
            jax.ShapeDtypeStruct((1, DE), F32),
            jax.ShapeDtypeStruct((1, DN), F32),
            jax.ShapeDtypeStruct((DN, DN), F32),
        ],
    )(v, W_edge, g, b_edge, W_node, b_node, jnp.asarray(BD_MASK))


_TXB = 2000


def _txin_body(xt_ref, out_ref):
    xt = xt_ref[...]
    out_ref[...] = xt.reshape(16, _TXB, 8).transpose(1, 2, 0).reshape(
        _TXB, 128)


def _txin(eft):
    grid = (E // 8 // _TXB,)
    return pl.pallas_call(
        _txin_body,
        grid=grid,
        in_specs=[pl.BlockSpec((DE, 8 * _TXB), lambda i: (0, i))],
        out_specs=pl.BlockSpec((_TXB, 128), lambda i: (i, 0)),
        out_shape=jax.ShapeDtypeStruct((E // 8, 128), F32),
    )(eft)


def _txout_body(y_ref, out_ref):
    y = y_ref[...]
    out_ref[...] = y.reshape(_TXB, 8, 16).transpose(2, 0, 1).reshape(
        16, 8 * _TXB)


def _txout(EN):
    grid = (E // 8 // _TXB,)
    return pl.pallas_call(
        _txout_body,
        grid=grid,
        in_specs=[pl.BlockSpec((_TXB, 128), lambda i: (i, 0))],
        out_specs=pl.BlockSpec((DE, 8 * _TXB), lambda i: (0, i)),
        out_shape=jax.ShapeDtypeStruct((DE, E), F32),
    )(EN)


def _g1_body(ps_hbm, pr_hbm, sidx_hbm, didx_hbm, out_hbm,
             si0, si1, di0, di1, rs0, rs1, rd0, rd1,
             sem0, sem1, wsem0, wsem1):
    w = lax.axis_index("s") * NC + lax.axis_index("c")
    ebase = w * EPW
    bufs = ((si0, di0, rs0, rd0, sem0, wsem0),
            (si1, di1, rs1, rd1, sem1, wsem1))

    def out_slice(c):
        return out_hbm.at[pl.ds((ebase + c * G1_CB) // G1_SUB, G1_NSUB)]

    def start(c, si, di, rs, rd, sem, wsem):
        row0 = (ebase + c * G1_CB) // G1_SUB
        pltpu.sync_copy(sidx_hbm.at[pl.ds(row0, G1_NSUB)], si)
        pltpu.sync_copy(didx_hbm.at[pl.ds(row0, G1_NSUB)], di)
        if c >= 2:
            pltpu.make_async_copy(rs, out_slice(c - 2), wsem).wait()
        for j in range(G1_NSUB):
            pltpu.make_async_copy(ps_hbm.at[si.at[j]], rs.at[j], sem).start()
            pltpu.make_async_copy(pr_hbm.at[di.at[j]], rd.at[j], sem).start()

    def drain(si, di, rs, rd, sem, wsem):
        for j in range(G1_NSUB):
            pltpu.make_async_copy(ps_hbm.at[si.at[j]], rs.at[j], sem).wait()
            pltpu.make_async_copy(pr_hbm.at[di.at[j]], rd.at[j], sem).wait()

    def process(c, rs, rd, wsem):
        def sub(j, carry):
            for i in range(G1_SUB):
                rs[j, i] = rs[j, i] + rd[j, i]
            return carry

        lax.fori_loop(0, G1_NSUB, sub, 0)
        if c < G1_NCH - 2:
            pltpu.make_async_copy(rs, out_slice(c), wsem).start()
        else:
            pltpu.sync_copy(rs, out_slice(c))

    start(0, *bufs[0])
    for c in range(G1_NCH):
        b = bufs[c % 2]
        if c + 1 < G1_NCH:
            start(c + 1, *bufs[(c + 1) % 2])
        drain(*b)
        process(c, b[2], b[3], b[5])


def _g1(P_s, P_r, sidx, didx):
    mesh = plsc.VectorSubcoreMesh(
        core_axis_name="c", subcore_axis_name="s", num_cores=NC,
        num_subcores=NS)
    f = functools.partial(
        pl.kernel,
        out_type=jax.ShapeDtypeStruct((E // G1_SUB, G1_SUB, DE), F32),
        mesh=mesh,
        compiler_params=pltpu.CompilerParams(use_tc_tiling_on_sc=False),
        scratch_types=[
            pltpu.VMEM((G1_NSUB, G1_SUB), jnp.int32),
            pltpu.VMEM((G1_NSUB, G1_SUB), jnp.int32),
            pltpu.VMEM((G1_NSUB, G1_SUB), jnp.int32),
            pltpu.VMEM((G1_NSUB, G1_SUB), jnp.int32),
            pltpu.VMEM((G1_NSUB, G1_SUB, DE), F32),
            pltpu.VMEM((G1_NSUB, G1_SUB, DE), F32),
            pltpu.VMEM((G1_NSUB, G1_SUB, DE), F32),
            pltpu.VMEM((G1_NSUB, G1_SUB, DE), F32),
            pltpu.SemaphoreType.DMA,
            pltpu.SemaphoreType.DMA,
            pltpu.SemaphoreType.DMA,
            pltpu.SemaphoreType.DMA,
        ],
    )(_g1_body)
    return f(P_s, P_r, sidx, didx)


def _edge_body(x_ref, gx_ref, bdw_ref, bdm_ref, ce_ref, ga_ref, be_ref,
               out_ref, acc_ref):
    x = x_ref[...]
    ce = jnp.concatenate([ce_ref[...]] * 8, axis=1)
    ga = jnp.concatenate([ga_ref[...]] * 8, axis=1)
    be = jnp.concatenate([be_ref[...]] * 8, axis=1)
    u = jnp.maximum(_dot(x, bdw_ref[...]) + gx_ref[...] + ce, 0.0)
    h = u + x
    mu = _dot(h, bdm_ref[...])
    r = h - mu
    var = _dot(r * r, bdm_ref[...])
    y = r * lax.rsqrt(var + 1e-3) * ga + be
    out_ref[...] = y

    @pl.when(pl.program_id(0) == 0)
    def _():
        acc_ref[...] = jnp.zeros_like(acc_ref)

    acc_ref[...] += jnp.sum(y, axis=0, keepdims=True)


def _edge(X, GX, BD_W, c_e, ln_g, ln_b):
    E8 = E // 8
    blk = 2000
    grid = (E8 // blk,)
    return pl.pallas_call(
        _edge_body,
        grid=grid,
        in_specs=[
            pl.BlockSpec((blk, 128), lambda i: (i, 0)),
            pl.BlockSpec((blk, 128), lambda i: (i, 0)),
            pl.BlockSpec((128, 128), lambda i: (0, 0)),
            pl.BlockSpec((128, 128), lambda i: (0, 0)),
            pl.BlockSpec((1, DE), lambda i: (0, 0)),
            pl.BlockSpec((1, DE), lambda i: (0, 0)),
            pl.BlockSpec((1, DE), lambda i: (0, 0)),
        ],
        out_specs=[
            pl.BlockSpec((blk, 128), lambda i: (i, 0)),
            pl.BlockSpec((1, 128), lambda i: (0, 0)),
        ],
        out_shape=[
            jax.ShapeDtypeStruct((E8, 128), F32),
            jax.ShapeDtypeStruct((1, 128), F32),
        ],
    )(X, GX, BD_W, jnp.asarray(BD_MEAN), c_e, ln_g, ln_b)


def _g2_body(en_hbm, cidx_hbm, out_hbm,
             idx0, idx1, rows0, rows1, ostage, sem0, sem1):
    w = lax.axis_index("s") * NC + lax.axis_index("c")
    ibase = w * NPW * K
    nch = jnp.where(w == NW - 1, (N - (NW - 1) * NPW) // G2_NPC,
                    NPW // G2_NPC)
    npairs = nch // 2
    bufs = ((idx0, rows0, sem0), (idx1, rows1, sem1))

    def start(c, idx_v, rows_v, sem):
        pltpu.sync_copy(
            cidx_hbm.at[pl.ds((ibase + c * G2_IDXC) // G2_SUB, G2_NSUB)],
            idx_v)
        for j in range(G2_NSUB):
            pltpu.make_async_copy(
                en_hbm.at[idx_v.at[j]], rows_v.at[j], sem).start()

    def drain(idx_v, rows_v, sem):
        for j in range(G2_NSUB):
            pltpu.make_async_copy(
                en_hbm.at[idx_v.at[j]], rows_v.at[j], sem).wait()

    def process(c, rows_v):
        npj = G2_SUB // K
        for k in range(G2_NPC):
            j, o = k // npj, (k % npj) * K
            vals = [rows_v[j, o + i] for i in range(K)]
            while len(vals) > 1:
                vals = [vals[i] + vals[i + 1] for i in range(0, len(vals), 2)]
            ostage[c * G2_NPC + k] = vals[0]

    start(0, *bufs[0])

    def body(j, carry):
        c0 = 2 * j
        c1 = c0 + 1
        start(c1, *bufs[1])
        drain(*bufs[0])
        process(c0, bufs[0][1])

        @pl.when(c0 + 2 < nch)
        def _():
            start(c0 + 2, *bufs[0])

        drain(*bufs[1])
        process(c1, bufs[1][1])
        return carry

    lax.fori_loop(0, npairs, body, 0)
    pltpu.sync_copy(ostage, out_hbm.at[pl.ds(w * NPW, NPW)])


def _g2(EN_flat, cidx):
    mesh = plsc.VectorSubcoreMesh(
        core_axis_name="c", subcore_axis_name="s", num_cores=NC,
        num_subcores=NS)
    f = functools.partial(
        pl.kernel,
        out_type=jax.ShapeDtypeStruct((NPAD, DE), F32),
        mesh=mesh,
        compiler_params=pltpu.CompilerParams(use_tc_tiling_on_sc=False),
        scratch_types=[
            pltpu.VMEM((G2_NSUB, G2_SUB), jnp.int32),
            pltpu.VMEM((G2_NSUB, G2_SUB), jnp.int32),
            pltpu.VMEM((G2_NSUB, G2_SUB, DE), F32),
            pltpu.VMEM((G2_NSUB, G2_SUB, DE), F32),
            pltpu.VMEM((NPW, DE), F32),
            pltpu.SemaphoreType.DMA,
            pltpu.SemaphoreType.DMA,
        ],
    )(_g2_body)
    return f(EN_flat, cidx)


def _node_body(v_ref, conn_ref, vl_ref, wn_ref, cn_ref, ga_ref, be_ref,
               m_ref, out_ref, acc_ref):
    v = v_ref[...]
    blk = v.shape[0]
    conn = conn_ref[...] / jnp.maximum(vl_ref[...], 1.0)
    pre = (_dot(v, wn_ref[0:128, :]) + _dot(conn, wn_ref[128:144, :])
           + cn_ref[...])
    u = jnp.maximum(pre, 0.0)
    h = u + v
    mu = _dot(h, m_ref[...])
    r = h - mu
    var = _dot(r * r, m_ref[...])
    y = r * lax.rsqrt(var + 1e-3) * ga_ref[...] + be_ref[...]
    out_ref[...] = y

    @pl.when(pl.program_id(0) == 0)
    def _():
        acc_ref[...] = jnp.zeros_like(acc_ref)

    row = (pl.program_id(0) * blk
           + lax.broadcasted_iota(jnp.int32, (blk, 1), 0))
    acc_ref[...] += jnp.sum(jnp.where(row < N, y, 0.0), axis=0,
                            keepdims=True)


def _node(v, conn, vl, W_node, c_n, ln_g, ln_b):
    blk = 2048
    grid = (NPAD // blk,)
    return pl.pallas_call(
        _node_body,
        grid=grid,
        in_specs=[
            pl.BlockSpec((blk, DN), lambda i: (i, 0)),
            pl.BlockSpec((blk, DE), lambda i: (i, 0)),
            pl.BlockSpec((blk, 1), lambda i: (i, 0)),
            pl.BlockSpec((272, DN), lambda i: (0, 0)),
            pl.BlockSpec((1, DN), lambda i: (0, 0)),
            pl.BlockSpec((1, DN), lambda i: (0, 0)),
            pl.BlockSpec((1, DN), lambda i: (0, 0)),
            pl.BlockSpec((DN, DN), lambda i: (0, 0)),
        ],
        out_specs=[
            pl.BlockSpec((blk, DN), lambda i: (i, 0)),
            pl.BlockSpec((1, DN), lambda i: (0, 0)),
        ],
        out_shape=[
            jax.ShapeDtypeStruct((N, DN), F32),
            jax.ShapeDtypeStruct((1, DN), F32),
        ],
    )(v, conn, vl, W_node, c_n, ln_g, ln_b, jnp.asarray(M_MEAN128))


def _glob_body(g_ref, av_ref, ae_ref, vn_ref, ve_ref, wg_ref, bg_ref,
               bng_ref, bnb_ref, bnm_ref, bnv_ref, fold_ref, out_ref):
    g = g_ref[...]
    agg_v = av_ref[...] / jnp.maximum(vn_ref[...], 1.0)
    agg_e = _dot(ae_ref[...], fold_ref[...]) / jnp.maximum(ve_ref[...], 1.0)
    x = (_dot(g, wg_ref[0:128, :]) + _dot(agg_v, wg_ref[128:256, :])
         + _dot(agg_e, wg_ref[256:272, :]) + bg_ref[...])
    u = jnp.maximum(x, 0.0)
    h = u + g
    out_ref[...] = ((h - bnm_ref[...]) * lax.rsqrt(bnv_ref[...] + 1e-3)
                    * bng_ref[...] + bnb_ref[...])


def _glob(g, acc_v, acc_e, vn, ve, W_glob, b_glob, bng, bnb, bnm, bnv):
    specs = [
        pl.BlockSpec((1, DG), lambda: (0, 0)),
        pl.BlockSpec((1, DN), lambda: (0, 0)),
        pl.BlockSpec((1, 128), lambda: (0, 0)),
        pl.BlockSpec((1, 1), lambda: (0, 0)),
        pl.BlockSpec((1, 1), lambda: (0, 0)),
        pl.BlockSpec((272, DG), lambda: (0, 0)),
        pl.BlockSpec((1, DG), lambda: (0, 0)),
        pl.BlockSpec((1, DG), lambda: (0, 0)),
        pl.BlockSpec((1, DG), lambda: (0, 0)),
        pl.BlockSpec((1, DG), lambda: (0, 0)),
        pl.BlockSpec((1, DG), lambda: (0, 0)),
        pl.BlockSpec((128, DE), lambda: (0, 0)),
    ]
    return pl.pallas_call(
        _glob_body,
        in_specs=specs,
        out_specs=pl.BlockSpec((1, DG), lambda: (0, 0)),
        out_shape=jax.ShapeDtypeStruct((1, DG), F32),
    )(g, acc_v, acc_e, vn, ve, W_glob, b_glob, bng, bnb, bnm, bnv,
      jnp.asarray(FOLD16))


def kernel(vertex_feat, edges_feat, global_feat, edges_idx,
           connected_edges_idx, valid_lens, valid_nodes, valid_edges,
           W_edge, b_edge, ln_e_gamma, ln_e_beta, W_node, b_node,
           ln_n_gamma, ln_n_beta, W_glob, b_glob, bn_gamma, bn_beta,
           bn_mean, bn_var):
    v = vertex_feat[0]
    g = global_feat
    eft = edges_feat[0].T
    sidx = edges_idx[0, :, 0].reshape(E // G1_SUB, G1_SUB)
    didx = edges_idx[0, :, 1].reshape(E // G1_SUB, G1_SUB)
    cidx = connected_edges_idx.reshape(N * K // G2_SUB, G2_SUB)
    vl = valid_lens.reshape(N, 1).astype(F32)
    vn = valid_nodes.reshape(1, 1).astype(F32)
    ve = valid_edges.reshape(1, 1).astype(F32)

    P_s, P_r, c_e, c_n, BD_W = _prep(
        v, W_edge, g, b_edge.reshape(1, DE), W_node, b_node.reshape(1, DN))

    ge = _g1(P_s, P_r, sidx, didx)
    X = edges_feat.reshape(E // 8, 128)

    GX = ge.reshape(E // 8, 128)
    EN, acc_e = _edge(X, GX, BD_W, c_e,
                      ln_e_gamma.reshape(1, DE), ln_e_beta.reshape(1, DE))

    conn = _g2(EN.reshape(E, DE), cidx)

    VN, acc_v = _node(v, conn, vl, W_node, c_n,
                      ln_n_gamma.reshape(1, DN), ln_n_beta.reshape(1, DN))

    gout = _glob(g, acc_v, acc_e, vn, ve, W_glob, b_glob.reshape(1, DG),
                 bn_gamma.reshape(1, DG), bn_beta.reshape(1, DG),
                 bn_mean.reshape(1, DG), bn_var.reshape(1, DG))

    return (VN.reshape(1, N, DN), EN.reshape(1, E, DE), gout)


# speedup vs baseline: 1.0029x; 1.0029x over previous
import functools

import numpy as np
import jax
import jax.numpy as jnp
from jax import lax
from jax.experimental import pallas as pl
from jax.experimental.pallas import tpu as pltpu
from jax.experimental.pallas import tpu_sc as plsc

N, E, K = 10000, 320000, 64
DN, DE, DG = 128, 16, 128
F32 = jnp.float32

NC, NS = 2, 16
NW = NC * NS

EPW = E // NW
G1_CB = 1000
G1_SUB = 40
G1_NSUB = G1_CB // G1_SUB
G1_NCH = EPW // G1_CB

NPAD = 10240
NPW = NPAD // NW
G2_NPC = 8
G2_IDXC = G2_NPC * K
G2_SUB = 128
G2_NSUB = G2_IDXC // G2_SUB

_PREC = lax.Precision.DEFAULT

_I8 = np.eye(8, dtype=np.float32)
BD_MASK = np.kron(_I8, np.ones((16, 16), np.float32))
BD_MEAN = np.kron(_I8, np.full((16, 16), 1.0 / 16, np.float32))
M_MEAN128 = np.full((128, 128), 1.0 / 128, np.float32)
FOLD16 = np.tile(np.eye(16, dtype=np.float32), (8, 1))


def _dot(a, b):
    return jnp.dot(a, b, preferred_element_type=F32, precision=_PREC)


def _prep_body(v_ref, we_ref, g_ref, be_ref, wn_ref, bn_ref, mask_ref,
               ps_ref, pr_ref, ce_ref, cn_ref, bdw_ref):
    v = v_ref[...]
    ps_ref[...] = _dot(v, we_ref[16:144, :])
    pr_ref[...] = _dot(v, we_ref[144:272, :])
    ce_ref[...] = _dot(g_ref[...], we_ref[272:400, :]) + be_ref[...]
    cn_ref[...] = _dot(g_ref[...], wn_ref[144:272, :]) + bn_ref[...]
    wee = we_ref[0:16, :]
    col = jnp.concatenate([wee] * 8, axis=0)
    bdw_ref[...] = jnp.concatenate([col] * 8, axis=1) * mask_ref[...]


def _prep(v, W_edge, g, b_edge, W_node, b_node):
    blk = 2000
    grid = (N // blk,)
    return pl.pallas_call(
        _prep_body,
        grid=grid,
        in_specs=[
            pl.BlockSpec((blk, DN), lambda t: (t, 0)),
            pl.BlockSpec((400, DE), lambda t: (0, 0)),
            pl.BlockSpec((1, DG), lambda t: (0, 0)),
            pl.BlockSpec((1, DE), lambda t: (0, 0)),
            pl.BlockSpec((272, DN), lambda t: (0, 0)),
            pl.BlockSpec((1, DN), lambda t: (0, 0)),
            pl.BlockSpec((DN, DN), lambda t: (0, 0)),
        ],
        out_specs=[
            pl.BlockSpec((blk, DE), lambda t: (t, 0)),
            pl.BlockSpec((blk, DE), lambda t: (t, 0)),
            pl.BlockSpec((1, DE), lambda t: (0, 0)),
            pl.BlockSpec((1, DN), lambda t: (0, 0)),
            pl.BlockSpec((DN, DN), lambda t: (0, 0)),
        ],
        out_shape=[
            jax.ShapeDtypeStruct((N, DE), F32),
            jax.ShapeDtypeStruct((N, DE), F32),
            jax.ShapeDtypeStruct((1, DE), F32),
            jax.ShapeDtypeStruct((1, DN), F32),
            jax.ShapeDtypeStruct((DN, DN), F32),
        ],
    )(v, W_edge, g, b_edge, W_node, b_node, jnp.asarray(BD_MASK))


def _g1_body(ps_hbm, pr_hbm, sidx_hbm, didx_hbm, out_hbm,
             si0, si1, di0, di1, rs0, rs1, rd0, rd1,
             sem0, sem1, wsem0, wsem1):
    w = lax.axis_index("s") * NC + lax.axis_index("c")
    ebase = w * EPW
    bufs = ((si0, di0, rs0, rd0, sem0, wsem0),
            (si1, di1, rs1, rd1, sem1, wsem1))

    def out_slice(c):
        return out_hbm.at[pl.ds((ebase + c * G1_CB) // G1_SUB, G1_NSUB)]

    def start(c, si, di, rs, rd, sem, wsem):
        row0 = (ebase + c * G1_CB) // G1_SUB
        pltpu.sync_copy(sidx_hbm.at[pl.ds(row0, G1_NSUB)], si)
        pltpu.sync_copy(didx_hbm.at[pl.ds(row0, G1_NSUB)], di)
        if c >= 2:
            pltpu.make_async_copy(rs, out_slice(c - 2), wsem).wait()
        for j in range(G1_NSUB):
            pltpu.make_async_copy(ps_hbm.at[si.at[j]], rs.at[j], sem).start()
            pltpu.make_async_copy(pr_hbm.at[di.at[j]], rd.at[j], sem).start()

    def drain(si, di, rs, rd, sem, wsem):
        for j in range(G1_NSUB):
            pltpu.make_async_copy(ps_hbm.at[si.at[j]], rs.at[j], sem).wait()
            pltpu.make_async_copy(pr_hbm.at[di.at[j]], rd.at[j], sem).wait()

    def process(c, rs, rd, wsem):
        def sub(j, carry):
            for i in range(G1_SUB):
                rs[j, i] = rs[j, i] + rd[j, i]
            return carry

        lax.fori_loop(0, G1_NSUB, sub, 0)
        if c < G1_NCH - 2:
            pltpu.make_async_copy(rs, out_slice(c), wsem).start()
        else:
            pltpu.sync_copy(rs, out_slice(c))

    start(0, *bufs[0])
    for c in range(G1_NCH):
        b = bufs[c % 2]
        if c + 1 < G1_NCH:
            start(c + 1, *bufs[(c + 1) % 2])
        drain(*b)
        process(c, b[2], b[3], b[5])


def _g1(P_s, P_r, sidx, didx):
    mesh = plsc.VectorSubcoreMesh(
        core_axis_name="c", subcore_axis_name="s", num_cores=NC,
        num_subcores=NS)
    f = functools.partial(
        pl.kernel,
        out_type=jax.ShapeDtypeStruct((E // G1_SUB, G1_SUB, DE), F32),
        mesh=mesh,
        compiler_params=pltpu.CompilerParams(use_tc_tiling_on_sc=False),
        scratch_types=[
            pltpu.VMEM((G1_NSUB, G1_SUB), jnp.int32),
            pltpu.VMEM((G1_NSUB, G1_SUB), jnp.int32),
            pltpu.VMEM((G1_NSUB, G1_SUB), jnp.int32),
            pltpu.VMEM((G1_NSUB, G1_SUB), jnp.int32),
            pltpu.VMEM((G1_NSUB, G1_SUB, DE), F32),
            pltpu.VMEM((G1_NSUB, G1_SUB, DE), F32),
            pltpu.VMEM((G1_NSUB, G1_SUB, DE), F32),
            pltpu.VMEM((G1_NSUB, G1_SUB, DE), F32),
            pltpu.SemaphoreType.DMA,
            pltpu.SemaphoreType.DMA,
            pltpu.SemaphoreType.DMA,
            pltpu.SemaphoreType.DMA,
        ],
    )(_g1_body)
    return f(P_s, P_r, sidx, didx)


def _edge_body(x_ref, gx_ref, bdw_ref, bdm_ref, ce_ref, ga_ref, be_ref,
               out_ref, acc_ref):
    x = x_ref[...]
    ce = jnp.concatenate([ce_ref[...]] * 8, axis=1)
    ga = jnp.concatenate([ga_ref[...]] * 8, axis=1)
    be = jnp.concatenate([be_ref[...]] * 8, axis=1)
    u = jnp.maximum(_dot(x, bdw_ref[...]) + gx_ref[...] + ce, 0.0)
    h = u + x
    mu = _dot(h, bdm_ref[...])
    r = h - mu
    var = _dot(r * r, bdm_ref[...])
    y = r * lax.rsqrt(var + 1e-3) * ga + be
    out_ref[...] = y

    @pl.when(pl.program_id(0) == 0)
    def _():
        acc_ref[...] = jnp.zeros_like(acc_ref)

    acc_ref[...] += jnp.sum(y, axis=0, keepdims=True)


def _edge(X, GX, BD_W, c_e, ln_g, ln_b):
    E8 = E // 8
    blk = 2000
    grid = (E8 // blk,)
    return pl.pallas_call(
        _edge_body,
        grid=grid,
        in_specs=[
            pl.BlockSpec((blk, 128), lambda i: (i, 0)),
            pl.BlockSpec((blk, 128), lambda i: (i, 0)),
            pl.BlockSpec((128, 128), lambda i: (0, 0)),
            pl.BlockSpec((128, 128), lambda i: (0, 0)),
            pl.BlockSpec((1, DE), lambda i: (0, 0)),
            pl.BlockSpec((1, DE), lambda i: (0, 0)),
            pl.BlockSpec((1, DE), lambda i: (0, 0)),
        ],
        out_specs=[
            pl.BlockSpec((blk, 128), lambda i: (i, 0)),
            pl.BlockSpec((1, 128), lambda i: (0, 0)),
        ],
        out_shape=[
            jax.ShapeDtypeStruct((E8, 128), F32),
            jax.ShapeDtypeStruct((1, 128), F32),
        ],
    )(X, GX, BD_W, jnp.asarray(BD_MEAN), c_e, ln_g, ln_b)


def _g2_body(en_hbm, cidx_hbm, out_hbm,
             idx0, idx1, rows0, rows1, ostage, sem0, sem1):
    w = lax.axis_index("s") * NC + lax.axis_index("c")
    ibase = w * NPW * K
    nch = jnp.where(w == NW - 1, (N - (NW - 1) * NPW) // G2_NPC,
                    NPW // G2_NPC)
    npairs = nch // 2
    bufs = ((idx0, rows0, sem0), (idx1, rows1, sem1))

    def start(c, idx_v, rows_v, sem):
        pltpu.sync_copy(
            cidx_hbm.at[pl.ds((ibase + c * G2_IDXC) // G2_SUB, G2_NSUB)],
            idx_v)
        for j in range(G2_NSUB):
            pltpu.make_async_copy(
                en_hbm.at[idx_v.at[j]], rows_v.at[j], sem).start()

    def drain(idx_v, rows_v, sem):
        for j in range(G2_NSUB):
            pltpu.make_async_copy(
                en_hbm.at[idx_v.at[j]], rows_v.at[j], sem).wait()

    def process(c, rows_v):
        npj = G2_SUB // K
        for k in range(G2_NPC):
            j, o = k // npj, (k % npj) * K
            vals = [rows_v[j, o + i] for i in range(K)]
            while len(vals) > 1:
                vals = [vals[i] + vals[i + 1] for i in range(0, len(vals), 2)]
            ostage[c * G2_NPC + k] = vals[0]

    start(0, *bufs[0])

    def body(j, carry):
        c0 = 2 * j
        c1 = c0 + 1
        start(c1, *bufs[1])
        drain(*bufs[0])
        process(c0, bufs[0][1])

        @pl.when(c0 + 2 < nch)
        def _():
            start(c0 + 2, *bufs[0])

        drain(*bufs[1])
        process(c1, bufs[1][1])
        return carry

    lax.fori_loop(0, npairs, body, 0)
    pltpu.sync_copy(ostage, out_hbm.at[pl.ds(w * NPW, NPW)])


def _g2(EN_flat, cidx):
    mesh = plsc.VectorSubcoreMesh(
        core_axis_name="c", subcore_axis_name="s", num_cores=NC,
        num_subcores=NS)
    f = functools.partial(
        pl.kernel,
        out_type=jax.ShapeDtypeStruct((NPAD, DE), F32),
        mesh=mesh,
        compiler_params=pltpu.CompilerParams(use_tc_tiling_on_sc=False),
        scratch_types=[
            pltpu.VMEM((G2_NSUB, G2_SUB), jnp.int32),
            pltpu.VMEM((G2_NSUB, G2_SUB), jnp.int32),
            pltpu.VMEM((G2_NSUB, G2_SUB, DE), F32),
            pltpu.VMEM((G2_NSUB, G2_SUB, DE), F32),
            pltpu.VMEM((NPW, DE), F32),
            pltpu.SemaphoreType.DMA,
            pltpu.SemaphoreType.DMA,
        ],
    )(_g2_body)
    return f(EN_flat, cidx)


def _node_body(v_ref, conn_ref, vl_ref, wn_ref, cn_ref, ga_ref, be_ref,
               m_ref, out_ref, acc_ref):
    v = v_ref[...]
    blk = v.shape[0]
    conn = conn_ref[...] / jnp.maximum(vl_ref[...], 1.0)
    pre = (_dot(v, wn_ref[0:128, :]) + _dot(conn, wn_ref[128:144, :])
           + cn_ref[...])
    u = jnp.maximum(pre, 0.0)
    h = u + v
    mu = _dot(h, m_ref[...])
    r = h - mu
    var = _dot(r * r, m_ref[...])
    y = r * lax.rsqrt(var + 1e-3) * ga_ref[...] + be_ref[...]
    out_ref[...] = y

    @pl.when(pl.program_id(0) == 0)
    def _():
        acc_ref[...] = jnp.zeros_like(acc_ref)

    row = (pl.program_id(0) * blk
           + lax.broadcasted_iota(jnp.int32, (blk, 1), 0))
    acc_ref[...] += jnp.sum(jnp.where(row < N, y, 0.0), axis=0,
                            keepdims=True)


def _node(v, conn, vl, W_node, c_n, ln_g, ln_b):
    blk = 2048
    grid = (NPAD // blk,)
    return pl.pallas_call(
        _node_body,
        grid=grid,
        in_specs=[
            pl.BlockSpec((blk, DN), lambda i: (i, 0)),
            pl.BlockSpec((blk, DE), lambda i: (i, 0)),
            pl.BlockSpec((blk, 1), lambda i: (i, 0)),
            pl.BlockSpec((272, DN), lambda i: (0, 0)),
            pl.BlockSpec((1, DN), lambda i: (0, 0)),
            pl.BlockSpec((1, DN), lambda i: (0, 0)),
            pl.BlockSpec((1, DN), lambda i: (0, 0)),
            pl.BlockSpec((DN, DN), lambda i: (0, 0)),
        ],
        out_specs=[
            pl.BlockSpec((blk, DN), lambda i: (i, 0)),
            pl.BlockSpec((1, DN), lambda i: (0, 0)),
        ],
        out_shape=[
            jax.ShapeDtypeStruct((N, DN), F32),
            jax.ShapeDtypeStruct((1, DN), F32),
        ],
    )(v, conn, vl, W_node, c_n, ln_g, ln_b, jnp.asarray(M_MEAN128))


def _glob_body(g_ref, av_ref, ae_ref, vn_ref, ve_ref, wg_ref, bg_ref,
               bng_ref, bnb_ref, bnm_ref, bnv_ref, fold_ref, out_ref):
    g = g_ref[...]
    agg_v = av_ref[...] / jnp.maximum(vn_ref[...], 1.0)
    agg_e = _dot(ae_ref[...], fold_ref[...]) / jnp.maximum(ve_ref[...], 1.0)
    x = (_dot(g, wg_ref[0:128, :]) + _dot(agg_v, wg_ref[128:256, :])
         + _dot(agg_e, wg_ref[256:272, :]) + bg_ref[...])
    u = jnp.maximum(x, 0.0)
    h = u + g
    out_ref[...] = ((h - bnm_ref[...]) * lax.rsqrt(bnv_ref[...] + 1e-3)
                    * bng_ref[...] + bnb_ref[...])


def _glob(g, acc_v, acc_e, vn, ve, W_glob, b_glob, bng, bnb, bnm, bnv):
    specs = [
        pl.BlockSpec((1, DG), lambda: (0, 0)),
        pl.BlockSpec((1, DN), lambda: (0, 0)),
        pl.BlockSpec((1, 128), lambda: (0, 0)),
        pl.BlockSpec((1, 1), lambda: (0, 0)),
        pl.BlockSpec((1, 1), lambda: (0, 0)),
        pl.BlockSpec((272, DG), lambda: (0, 0)),
        pl.BlockSpec((1, DG), lambda: (0, 0)),
        pl.BlockSpec((1, DG), lambda: (0, 0)),
        pl.BlockSpec((1, DG), lambda: (0, 0)),
        pl.BlockSpec((1, DG), lambda: (0, 0)),
        pl.BlockSpec((1, DG), lambda: (0, 0)),
        pl.BlockSpec((128, DE), lambda: (0, 0)),
    ]
    return pl.pallas_call(
        _glob_body,
        in_specs=specs,
        out_specs=pl.BlockSpec((1, DG), lambda: (0, 0)),
        out_shape=jax.ShapeDtypeStruct((1, DG), F32),
    )(g, acc_v, acc_e, vn, ve, W_glob, b_glob, bng, bnb, bnm, bnv,
      jnp.asarray(FOLD16))


def kernel(vertex_feat, edges_feat, global_feat, edges_idx,
           connected_edges_idx, valid_lens, valid_nodes, valid_edges,
           W_edge, b_edge, ln_e_gamma, ln_e_beta, W_node, b_node,
           ln_n_gamma, ln_n_beta, W_glob, b_glob, bn_gamma, bn_beta,
           bn_mean, bn_var):
    v = vertex_feat[0]
    g = global_feat
    sidx = edges_idx[0, :, 0].reshape(E // G1_SUB, G1_SUB)
    didx = edges_idx[0, :, 1].reshape(E // G1_SUB, G1_SUB)
    cidx = connected_edges_idx.reshape(N * K // G2_SUB, G2_SUB)
    vl = valid_lens.reshape(N, 1).astype(F32)
    vn = valid_nodes.reshape(1, 1).astype(F32)
    ve = valid_edges.reshape(1, 1).astype(F32)

    P_s, P_r, c_e, c_n, BD_W = _prep(
        v, W_edge, g, b_edge.reshape(1, DE), W_node, b_node.reshape(1, DN))

    ge = _g1(P_s, P_r, sidx, didx)
    X = edges_feat.reshape(E // 8, 128)

    GX = ge.reshape(E // 8, 128)
    EN, acc_e = _edge(X, GX, BD_W, c_e,
                      ln_e_gamma.reshape(1, DE), ln_e_beta.reshape(1, DE))

    conn = _g2(EN.reshape(E, DE), cidx)

    VN, acc_v = _node(v, conn, vl, W_node, c_n,
                      ln_n_gamma.reshape(1, DN), ln_n_beta.reshape(1, DN))

    gout = _glob(g, acc_v, acc_e, vn, ve, W_glob, b_glob.reshape(1, DG),
                 bn_gamma.reshape(1, DG), bn_beta.reshape(1, DG),
                 bn_mean.reshape(1, DG), bn_var.reshape(1, DG))

    return (VN.reshape(1, N, DN), EN.reshape(1, E, DE), gout)
